# baseline (speedup 1.0000x reference)
import jax
import jax.numpy as jnp
from jax import lax
from jax.experimental import pallas as pl
from jax.experimental.pallas import tpu as pltpu
from jax.experimental.pallas import tpu_sc as plsc

B = 8
N = 20000
C = 80
MAXDET = 100
NCHUNK = N // 16
SELPAD = 112
NFETCH = 104

_NEG = -1.0
_CONF = 0.5
_BIG = 1 << 30


def _scores_body(cls_ref, out_ref):
    s = jnp.max(cls_ref[...], axis=1, keepdims=True)
    out_ref[...] = jnp.where(s > _CONF, s, _NEG)


def _compute_scores(cls2d):
    rows = B * N
    blk = 2000
    return pl.pallas_call(
        _scores_body,
        grid=(rows // blk,),
        in_specs=[pl.BlockSpec((blk, C), lambda i: (i, 0))],
        out_specs=pl.BlockSpec((blk, 1), lambda i: (i, 0)),
        out_shape=jax.ShapeDtypeStruct((rows, 1), jnp.float32),
    )(cls2d)


def _sc_body(boxes_hbm, scores_hbm, cls_hbm, box_out, idx_out, cls_out,
             boxes_v, scores_v, sel_v, maskf_v, idx_v, rows_v, boxout_v,
             clsidx_v, st_v, sem):
    lanes = lax.broadcasted_iota(jnp.int32, (16,), 0)
    wid = lax.axis_index("s") * 2 + lax.axis_index("c")

    @pl.when(wid < B)
    def _run():
        b = wid
        lanes4 = lanes * 4
        coord = lanes & 3
        pltpu.sync_copy(boxes_hbm.at[b], boxes_v)
        pltpu.sync_copy(scores_hbm.at[b], scores_v)

        st_v[pl.ds(0, 16)] = jnp.where(lanes == 2, jnp.int32(-1),
                                       jnp.int32(0))

        def argmax_scores():
            def chunk(i, carry):
                best, besti = carry
                v = scores_v[pl.ds(i * 16, 16)]
                gi = i * 16 + lanes
                m = v > best
                return jnp.where(m, v, best), jnp.where(m, gi, besti)

            best, besti = lax.fori_loop(
                0, NCHUNK, chunk,
                (jnp.full((16,), -2.0, jnp.float32),
                 jnp.zeros((16,), jnp.int32)))
            mx = jnp.max(best)
            pick = jnp.min(jnp.where(best == mx, besti, _BIG))
            return mx, pick

        def nms_iter(_, c):
            st = st_v[pl.ds(0, 16)]
            k = st[0]
            done = st[1]

            @pl.when(done == 0)
            def _active():
                mx, pick = argmax_scores()
                have = mx > _CONF
                pickv = jnp.full((16,), pick, jnp.int32)
                pb = plsc.load_gather(boxes_v, [pick * 4 + coord])
                y1 = jnp.max(jnp.where(lanes == 0, pb, -9.0))
                x1 = jnp.max(jnp.where(lanes == 1, pb, -9.0))
                y2 = jnp.max(jnp.where(lanes == 2, pb, -9.0))
                x2 = jnp.max(jnp.where(lanes == 3, pb, -9.0))
                area = (jnp.maximum(y2 - y1, 0.0)
                        * jnp.maximum(x2 - x1, 0.0))
                degen = area == 0.0

                @pl.when(have & (~degen))
                def _select_and_suppress():
                    plsc.store_scatter(sel_v,
                                       [jnp.full((16,), k, jnp.int32)],
                                       pickv, mask=lanes == 0)

                    def sup(i, _):
                        ds = pl.ds(i * 16, 16)
                        il4 = i * 64 + lanes4
                        Y1 = plsc.load_gather(boxes_v, [il4])
                        X1 = plsc.load_gather(boxes_v, [il4 + 1])
                        Y2 = plsc.load_gather(boxes_v, [il4 + 2])
                        X2 = plsc.load_gather(boxes_v, [il4 + 3])
                        yy1 = jnp.maximum(y1, Y1)
                        xx1 = jnp.maximum(x1, X1)
                        yy2 = jnp.minimum(y2, Y2)
                        xx2 = jnp.minimum(x2, X2)
                        inter = (jnp.maximum(yy2 - yy1, 0.0)
                                 * jnp.maximum(xx2 - xx1, 0.0))
                        area2 = (jnp.maximum(Y2 - Y1, 0.0)
                                 * jnp.maximum(X2 - X1, 0.0))
                        union = area + area2 - inter
                        kill = (inter + inter) > union
                        scores_v[ds] = jnp.where(kill, _NEG, scores_v[ds])
                        return 0

                    lax.fori_loop(0, NCHUNK, sup, 0)

                cont = have & (~degen)
                k2 = jnp.where(cont, k + 1, k)
                done2 = jnp.where(cont, jnp.int32(0), jnp.int32(1))
                fill2 = jnp.where(have, jnp.where(degen, pick, -1),
                                  jnp.int32(-1))
                st_v[pl.ds(0, 16)] = jnp.where(
                    lanes == 0, k2, jnp.where(lanes == 1, done2, fill2))

            return c

        lax.fori_loop(0, MAXDET, nms_iter, 0)

        st = st_v[pl.ds(0, 16)]
        kfin = st[0]
        fill = st[2]

        def finalize(cc, _):
            ds = pl.ds(cc * 16, 16)
            gi = cc * 16 + lanes
            s = jnp.where(gi < kfin, sel_v[ds], fill)
            sel_v[ds] = s
            m = s >= 0
            maskf_v[ds] = jnp.where(m, 1.0, 0.0)
            idx_v[ds] = jnp.maximum(s, 0) + b * N
            return 0

        lax.fori_loop(0, SELPAD // 16, finalize, 0)

        def fire(r, _):
            gi = idx_v[pl.ds(r, 16)][0]
            pltpu.async_copy(cls_hbm.at[gi], rows_v.at[r], sem)
            return 0

        lax.fori_loop(0, NFETCH, fire, 0)
        pltpu.make_async_copy(cls_hbm.at[pl.ds(0, NFETCH)], rows_v,
                              sem).wait()

        def row_fn(r, _):
            mf = maskf_v[pl.ds(r, 16)][0]

            def cchunk(cc, carry):
                best, besti = carry
                ds = pl.ds(cc * 16, 16)
                v = rows_v[r, ds] * mf
                rows_v[r, ds] = v
                gi = cc * 16 + lanes
                m2 = v > best
                return jnp.where(m2, v, best), jnp.where(m2, gi, besti)

            best, besti = lax.fori_loop(
                0, C // 16, cchunk,
                (jnp.full((16,), -1.0, jnp.float32),
                 jnp.zeros((16,), jnp.int32)))
            mxv = jnp.max(best)
            am = jnp.min(jnp.where(best == mxv, besti, _BIG))
            plsc.store_scatter(clsidx_v, [jnp.full((16,), r, jnp.int32)],
                               jnp.full((16,), am, jnp.int32),
                               mask=lanes == 0)
            return 0

        lax.fori_loop(0, MAXDET, row_fn, 0)

        def bchunk(cc, _):
            rowidx = cc * 4 + (lanes >> 2)
            selc = plsc.load_gather(sel_v, [rowidx])
            mfc = plsc.load_gather(maskf_v, [rowidx])
            vals = plsc.load_gather(boxes_v,
                                    [jnp.maximum(selc, 0) * 4 + coord])
            plsc.store_scatter(boxout_v, [rowidx, coord], vals * mfc)
            return 0

        lax.fori_loop(0, MAXDET // 4, bchunk, 0)

        pltpu.sync_copy(rows_v.at[pl.ds(0, MAXDET)], cls_out.at[b])
        pltpu.sync_copy(boxout_v.at[pl.ds(0, MAXDET)], box_out.at[b])
        pltpu.sync_copy(clsidx_v, idx_out.at[b])


def _sc_nms(boxes_flat, scores, cls2d):
    mesh = plsc.VectorSubcoreMesh(core_axis_name="c", subcore_axis_name="s",
                                  num_cores=2, num_subcores=16)
    fn = pl.kernel(
        _sc_body,
        out_type=(
            jax.ShapeDtypeStruct((B, MAXDET, 4), jnp.float32),
            jax.ShapeDtypeStruct((B, 128), jnp.int32),
            jax.ShapeDtypeStruct((B, MAXDET, C), jnp.float32),
        ),
        mesh=mesh,
        scratch_types=[
            pltpu.VMEM((4 * N,), jnp.float32),
            pltpu.VMEM((N,), jnp.float32),
            pltpu.VMEM((SELPAD,), jnp.int32),
            pltpu.VMEM((128,), jnp.float32),
            pltpu.VMEM((128,), jnp.int32),
            pltpu.VMEM((NFETCH, C), jnp.float32),
            pltpu.VMEM((SELPAD, 4), jnp.float32),
            pltpu.VMEM((128,), jnp.int32),
            pltpu.VMEM((16,), jnp.int32),
            pltpu.SemaphoreType.DMA,
        ],
        compiler_params=pltpu.CompilerParams(needs_layout_passes=False),
    )
    return fn(boxes_flat, scores, cls2d)


def kernel(box_prediction, class_prediction):
    cls2d = class_prediction.reshape(B * N, C)
    scores = _compute_scores(cls2d).reshape(B, N)
    boxes_flat = box_prediction.reshape(B, N * 4)
    nms_box, cls_idx, nms_cls = _sc_nms(boxes_flat, scores, cls2d)
    cls_idx = cls_idx[:, :MAXDET].astype(jnp.int64)
    return nms_box, cls_idx, nms_cls


# speedup vs baseline: 1.5521x; 1.5521x over previous
import jax
import jax.numpy as jnp
from jax import lax
from jax.experimental import pallas as pl
from jax.experimental.pallas import tpu as pltpu
from jax.experimental.pallas import tpu_sc as plsc

B = 8
N = 20000
C = 80
MAXDET = 100
NCHUNK = N // 16
SELPAD = 112
NFETCH = 104

_NEG = -1.0
_CONF = 0.5
_BIG = 1 << 30


def _scores_body(cls_ref, out_ref):
    s = jnp.max(cls_ref[0], axis=1)
    out_ref[...] = jnp.where(s > _CONF, s, _NEG).reshape(1, 1, N)


def _compute_scores(cls3d):
    return pl.pallas_call(
        _scores_body,
        grid=(B,),
        in_specs=[pl.BlockSpec((1, N, C), lambda b: (b, 0, 0))],
        out_specs=pl.BlockSpec((1, 1, N), lambda b: (b, 0, 0)),
        out_shape=jax.ShapeDtypeStruct((B, 1, N), jnp.float32),
    )(cls3d)


def _sc_body(boxes_hbm, scores_hbm, cls_hbm, box_out, idx_out, cls_out,
             boxes_v, scores_v, sel_v, maskf_v, idx_v, rows_v, boxout_v,
             clsidx_v, st_v, sem):
    lanes = lax.broadcasted_iota(jnp.int32, (16,), 0)
    wid = lax.axis_index("s") * 2 + lax.axis_index("c")

    @pl.when(wid < B)
    def _run():
        b = wid
        lanes4 = lanes * 4
        coord = lanes & 3
        pltpu.sync_copy(boxes_hbm.at[b], boxes_v)
        pltpu.sync_copy(scores_hbm.at[b], scores_v)

        st_v[pl.ds(0, 16)] = jnp.where(lanes == 2, jnp.int32(-1),
                                       jnp.int32(0))

        def argmax_scores():
            def chunk(i, carry):
                best, besti = carry
                v = scores_v[0, pl.ds(i * 16, 16)]
                gi = i * 16 + lanes
                m = v > best
                return jnp.where(m, v, best), jnp.where(m, gi, besti)

            best, besti = lax.fori_loop(
                0, NCHUNK, chunk,
                (jnp.full((16,), -2.0, jnp.float32),
                 jnp.zeros((16,), jnp.int32)))
            mx = jnp.max(best)
            pick = jnp.min(jnp.where(best == mx, besti, _BIG))
            return mx, pick

        def nms_iter(_, c):
            st = st_v[pl.ds(0, 16)]
            k = st[0]
            done = st[1]

            @pl.when(done == 0)
            def _active():
                mx, pick = argmax_scores()
                have = mx > _CONF
                pickv = jnp.full((16,), pick, jnp.int32)
                pb = plsc.load_gather(boxes_v, [pick * 4 + coord])
                y1 = jnp.max(jnp.where(lanes == 0, pb, -9.0))
                x1 = jnp.max(jnp.where(lanes == 1, pb, -9.0))
                y2 = jnp.max(jnp.where(lanes == 2, pb, -9.0))
                x2 = jnp.max(jnp.where(lanes == 3, pb, -9.0))
                area = (jnp.maximum(y2 - y1, 0.0)
                        * jnp.maximum(x2 - x1, 0.0))
                degen = area == 0.0

                @pl.when(have & (~degen))
                def _select_and_suppress():
                    plsc.store_scatter(sel_v,
                                       [jnp.full((16,), k, jnp.int32)],
                                       pickv, mask=lanes == 0)

                    def sup(i, _):
                        ds = pl.ds(i * 16, 16)
                        il4 = i * 64 + lanes4
                        Y1 = plsc.load_gather(boxes_v, [il4])
                        X1 = plsc.load_gather(boxes_v, [il4 + 1])
                        Y2 = plsc.load_gather(boxes_v, [il4 + 2])
                        X2 = plsc.load_gather(boxes_v, [il4 + 3])
                        yy1 = jnp.maximum(y1, Y1)
                        xx1 = jnp.maximum(x1, X1)
                        yy2 = jnp.minimum(y2, Y2)
                        xx2 = jnp.minimum(x2, X2)
                        inter = (jnp.maximum(yy2 - yy1, 0.0)
                                 * jnp.maximum(xx2 - xx1, 0.0))
                        area2 = (jnp.maximum(Y2 - Y1, 0.0)
                                 * jnp.maximum(X2 - X1, 0.0))
                        union = area + area2 - inter
                        kill = (inter + inter) > union
                        scores_v[0, ds] = jnp.where(kill, _NEG, scores_v[0, ds])
                        return 0

                    lax.fori_loop(0, NCHUNK, sup, 0)

                cont = have & (~degen)
                k2 = jnp.where(cont, k + 1, k)
                done2 = jnp.where(cont, jnp.int32(0), jnp.int32(1))
                fill2 = jnp.where(have, jnp.where(degen, pick, -1),
                                  jnp.int32(-1))
                st_v[pl.ds(0, 16)] = jnp.where(
                    lanes == 0, k2, jnp.where(lanes == 1, done2, fill2))

            return c

        lax.fori_loop(0, MAXDET, nms_iter, 0)

        st = st_v[pl.ds(0, 16)]
        kfin = st[0]
        fill = st[2]

        def finalize(cc, _):
            ds = pl.ds(cc * 16, 16)
            gi = cc * 16 + lanes
            s = jnp.where(gi < kfin, sel_v[ds], fill)
            sel_v[ds] = s
            m = s >= 0
            maskf_v[ds] = jnp.where(m, 1.0, 0.0)
            idx_v[ds] = jnp.maximum(s, 0) + b * N
            return 0

        lax.fori_loop(0, SELPAD // 16, finalize, 0)

        def fire(r, _):
            gi = idx_v[pl.ds(r, 16)][0]
            pltpu.async_copy(cls_hbm.at[gi], rows_v.at[r], sem)
            return 0

        lax.fori_loop(0, NFETCH, fire, 0)
        pltpu.make_async_copy(cls_hbm.at[pl.ds(0, NFETCH)], rows_v,
                              sem).wait()

        def row_fn(r, _):
            mf = maskf_v[pl.ds(r, 16)][0]

            def cchunk(cc, carry):
                best, besti = carry
                ds = pl.ds(cc * 16, 16)
                v = rows_v[r, ds] * mf
                rows_v[r, ds] = v
                gi = cc * 16 + lanes
                m2 = v > best
                return jnp.where(m2, v, best), jnp.where(m2, gi, besti)

            best, besti = lax.fori_loop(
                0, C // 16, cchunk,
                (jnp.full((16,), -1.0, jnp.float32),
                 jnp.zeros((16,), jnp.int32)))
            mxv = jnp.max(best)
            am = jnp.min(jnp.where(best == mxv, besti, _BIG))
            plsc.store_scatter(clsidx_v, [jnp.full((16,), r, jnp.int32)],
                               jnp.full((16,), am, jnp.int32),
                               mask=lanes == 0)
            return 0

        lax.fori_loop(0, MAXDET, row_fn, 0)

        def bchunk(cc, _):
            rowidx = cc * 4 + (lanes >> 2)
            selc = plsc.load_gather(sel_v, [rowidx])
            mfc = plsc.load_gather(maskf_v, [rowidx])
            vals = plsc.load_gather(boxes_v,
                                    [jnp.maximum(selc, 0) * 4 + coord])
            plsc.store_scatter(boxout_v, [rowidx, coord], vals * mfc)
            return 0

        lax.fori_loop(0, MAXDET // 4, bchunk, 0)

        pltpu.sync_copy(rows_v.at[pl.ds(0, MAXDET)], cls_out.at[b])
        pltpu.sync_copy(boxout_v.at[pl.ds(0, MAXDET)], box_out.at[b])
        pltpu.sync_copy(clsidx_v, idx_out.at[b])


def _sc_nms(boxes_flat, scores, cls2d):
    mesh = plsc.VectorSubcoreMesh(core_axis_name="c", subcore_axis_name="s",
                                  num_cores=2, num_subcores=16)
    fn = pl.kernel(
        _sc_body,
        out_type=(
            jax.ShapeDtypeStruct((B, MAXDET, 4), jnp.float32),
            jax.ShapeDtypeStruct((B, 128), jnp.int32),
            jax.ShapeDtypeStruct((B, MAXDET, C), jnp.float32),
        ),
        mesh=mesh,
        scratch_types=[
            pltpu.VMEM((4 * N,), jnp.float32),
            pltpu.VMEM((1, N), jnp.float32),
            pltpu.VMEM((SELPAD,), jnp.int32),
            pltpu.VMEM((128,), jnp.float32),
            pltpu.VMEM((128,), jnp.int32),
            pltpu.VMEM((NFETCH, C), jnp.float32),
            pltpu.VMEM((SELPAD, 4), jnp.float32),
            pltpu.VMEM((128,), jnp.int32),
            pltpu.VMEM((16,), jnp.int32),
            pltpu.SemaphoreType.DMA,
        ],
        compiler_params=pltpu.CompilerParams(needs_layout_passes=False),
    )
    return fn(boxes_flat, scores, cls2d)


def kernel(box_prediction, class_prediction):
    cls2d = class_prediction.reshape(B * N, C)
    scores = _compute_scores(class_prediction)
    boxes_flat = box_prediction.reshape(B, N * 4)
    nms_box, cls_idx, nms_cls = _sc_nms(boxes_flat, scores, cls2d)
    cls_idx = cls_idx[:, :MAXDET].astype(jnp.int64)
    return nms_box, cls_idx, nms_cls
